# baseline (speedup 1.0000x reference)
import functools

import jax
import jax.numpy as jnp
from jax import lax
from jax.experimental import pallas as pl
from jax.experimental.pallas import tpu as pltpu
from jax.experimental.pallas import tpu_sc as plsc

B = 16384
L = 20
D = 32
V = 1000000
NC = 2
NS = 16
NW = NC * NS
PER_W = B // NW
C = 16
NCHUNK = PER_W // C
IDX_PER_CHUNK = C * L
SPAD = 32
RPS = 128 // D
OROWS = 8

lane16 = None


def _embed_body(
    idx_hbm, w_hbm, rcol_hbm, table_hbm, out_hbm,
    idx_v0, idx_v1, rows_v0, rows_v1, w_v0, w_v1, rc_v0, rc_v1,
    out_v, sem0, sem1,
):
    cid = lax.axis_index("c")
    sid = lax.axis_index("s")
    wid = cid * NS + sid

    idx_b = (idx_v0, idx_v1)
    rows_b = (rows_v0, rows_v1)
    w_b = (w_v0, w_v1)
    rc_b = (rc_v0, rc_v1)
    sems = (sem0, sem1)

    lane = lax.iota(jnp.int32, 16)

    g_slices = [(0, 128), (128, 128), (256, 64)]

    def stage(ci, b):
        pltpu.sync_copy(idx_hbm.at[wid, ci, 0], idx_b[b])
        for s0, sn in g_slices:
            pltpu.async_copy(
                table_hbm.at[idx_b[b].at[pl.ds(s0, sn)]],
                rows_b[b].at[pl.ds(s0, sn)],
                sems[b],
            )
        pltpu.sync_copy(w_hbm.at[wid, ci, 0], w_b[b])
        pltpu.sync_copy(rcol_hbm.at[wid, ci, 0], rc_b[b])

    def drain(b):
        for s0, sn in g_slices:
            pltpu.make_async_copy(
                table_hbm.at[idx_b[b].at[pl.ds(s0, sn)]],
                rows_b[b].at[pl.ds(s0, sn)],
                sems[b],
            ).wait()

    def combine(ci, b):
        rows_v = rows_b[b]
        w_v = w_b[b]
        rc_v = rc_b[b]

        def b_body(bb, carry):
            woff = pl.multiple_of(bb * SPAD, SPAD)
            w0 = w_v[pl.ds(woff, 16)]
            w1 = w_v[pl.ds(woff + 16, 16)]
            rc0 = rc_v[pl.ds(woff, 16)]
            rc1 = rc_v[pl.ds(woff + 16, 16)]
            acc = [jnp.zeros((16,), jnp.float32) for _ in range(4)]
            slot0 = bb * L
            for l in range(L):
                ws, rs, j = (w0, rc0, l) if l < 16 else (w1, rc1, l - 16)
                cidx = jnp.full((16,), j, jnp.int32)
                wl = ws.at[cidx].get(mode="promise_in_bounds")
                rc = rs.at[cidx].get(mode="promise_in_bounds")
                rsplat = jnp.full((16,), 0, jnp.int32) + (slot0 + l)
                col = rc + lane
                g0 = plsc.load_gather(rows_v, [rsplat, col])
                g1 = plsc.load_gather(rows_v, [rsplat, col + 16])
                p = 2 * (l % 2)
                acc[p] = acc[p] + wl * g0
                acc[p + 1] = acc[p + 1] + wl * g1
            obase = (ci % 2) * C + bb
            orow = jnp.full((16,), 0, jnp.int32) + (obase // RPS)
            ocol0 = (obase % RPS) * D
            plsc.store_scatter(out_v, [orow, ocol0 + lane], acc[0] + acc[2])
            plsc.store_scatter(out_v, [orow, ocol0 + 16 + lane], acc[1] + acc[3])
            return carry

        lax.fori_loop(0, C, b_body, 0)

        @pl.when(ci % 2 == 1)
        def _():
            base = pl.multiple_of(
                wid * (PER_W * D // 128) + (ci // 2) * OROWS, OROWS
            )
            pltpu.sync_copy(out_v, out_hbm.at[pl.ds(base, OROWS)])

    stage(0, 0)

    def outer(c2, carry):
        for b in range(2):
            ci = c2 * 2 + b

            @pl.when(ci + 1 < NCHUNK)
            def _():
                stage(ci + 1, 1 - b)

            drain(b)
            combine(ci, b)
        return carry

    lax.fori_loop(0, NCHUNK // 2, outer, 0)


SB = 256
NI = 977
P = NI * SB
VPAD = RPS * P


def _tr_body(x0, x1, x2, x3, out_ref):
    eye = jnp.eye(D, dtype=jnp.float32)
    dn = (((0,), (0,)), ((), ()))

    def tr(x):
        return jax.lax.dot_general(
            x[...], eye, dn,
            precision=jax.lax.Precision.HIGHEST,
            preferred_element_type=jnp.float32,
        )

    out_ref[...] = jnp.concatenate([tr(x0), tr(x1), tr(x2), tr(x3)], axis=1)


_tr_call = pl.pallas_call(
    _tr_body,
    grid=(NI,),
    in_specs=[
        pl.BlockSpec((D, SB), functools.partial(lambda j, i: (0, j * NI + i), j))
        for j in range(RPS)
    ],
    out_specs=pl.BlockSpec((SB, 128), lambda i: (i, 0)),
    out_shape=jax.ShapeDtypeStruct((P, 128), jnp.float32),
)


_embed_call = functools.partial(
    pl.kernel,
    mesh=plsc.VectorSubcoreMesh(core_axis_name="c", subcore_axis_name="s"),
    compiler_params=pltpu.CompilerParams(
        needs_layout_passes=False, use_tc_tiling_on_sc=True
    ),
    out_type=jax.ShapeDtypeStruct((B * D // 128, 128), jnp.float32),
    scratch_types=[
        pltpu.VMEM((IDX_PER_CHUNK,), jnp.int32),
        pltpu.VMEM((IDX_PER_CHUNK,), jnp.int32),
        pltpu.VMEM((IDX_PER_CHUNK, 128), jnp.float32),
        pltpu.VMEM((IDX_PER_CHUNK, 128), jnp.float32),
        pltpu.VMEM((C * SPAD,), jnp.float32),
        pltpu.VMEM((C * SPAD,), jnp.float32),
        pltpu.VMEM((C * SPAD,), jnp.int32),
        pltpu.VMEM((C * SPAD,), jnp.int32),
        pltpu.VMEM((OROWS, 128), jnp.float32),
        pltpu.SemaphoreType.DMA,
        pltpu.SemaphoreType.DMA,
    ],
)(_embed_body)


@jax.jit
def kernel(indices, weights, table):
    idx32 = indices.astype(jnp.int32)
    q = (idx32 % P).reshape(NW, NCHUNK, 1, IDX_PER_CHUNK)
    rc = jnp.pad((idx32 // P) * D, ((0, 0), (0, SPAD - L)))
    rc = rc.reshape(NW, NCHUNK, 1, C * SPAD)
    w = jnp.pad(weights.astype(jnp.float32), ((0, 0), (0, SPAD - L)))
    w = w.reshape(NW, NCHUNK, 1, C * SPAD)
    tt = jnp.pad(table, ((0, VPAD - V), (0, 0))).T
    tb = _tr_call(tt, tt, tt, tt)
    out = _embed_call(q, w, rc, tb)
    return out.reshape(B, D)


# speedup vs baseline: 1.4105x; 1.4105x over previous
import functools

import jax
import jax.numpy as jnp
from jax import lax
from jax.experimental import pallas as pl
from jax.experimental.pallas import tpu as pltpu
from jax.experimental.pallas import tpu_sc as plsc

B = 16384
L = 20
D = 32
V = 1000000
NC = 2
NS = 16
NW = NC * NS
PER_W = B // NW
C = 16
NCHUNK = PER_W // C
IDX_PER_CHUNK = C * L
SPAD = 32
RPS = 128 // D
OROWS = 8

lane16 = None


def _embed_body(
    idx_hbm, w_hbm, rcol_hbm, table_hbm, out_hbm,
    idx_v0, idx_v1, rows_v0, rows_v1, w_v0, w_v1, rc_v0, rc_v1,
    out_v, sem0, sem1,
):
    cid = lax.axis_index("c")
    sid = lax.axis_index("s")
    wid = cid * NS + sid

    idx_b = (idx_v0, idx_v1)
    rows_b = (rows_v0, rows_v1)
    w_b = (w_v0, w_v1)
    rc_b = (rc_v0, rc_v1)
    sems = (sem0, sem1)

    lane = lax.iota(jnp.int32, 16)

    g_slices = [(0, 128), (128, 128), (256, 64)]

    def stage(ci, b):
        pltpu.sync_copy(idx_hbm.at[wid, ci, 0], idx_b[b])
        for s0, sn in g_slices:
            pltpu.async_copy(
                table_hbm.at[idx_b[b].at[pl.ds(s0, sn)]],
                rows_b[b].at[pl.ds(s0, sn)],
                sems[b],
            )
        pltpu.sync_copy(w_hbm.at[wid, ci, 0], w_b[b])
        pltpu.sync_copy(rcol_hbm.at[wid, ci, 0], rc_b[b])

    def drain(b):
        for s0, sn in g_slices:
            pltpu.make_async_copy(
                table_hbm.at[idx_b[b].at[pl.ds(s0, sn)]],
                rows_b[b].at[pl.ds(s0, sn)],
                sems[b],
            ).wait()

    def combine(ci, b):
        rows_v = rows_b[b]
        w_v = w_b[b]
        rc_v = rc_b[b]

        def b_body(bb, carry):
            woff = pl.multiple_of(bb * SPAD, SPAD)
            w0 = w_v[pl.ds(woff, 16)]
            w1 = w_v[pl.ds(woff + 16, 16)]
            rc0 = rc_v[pl.ds(woff, 16)]
            rc1 = rc_v[pl.ds(woff + 16, 16)]
            acc = [jnp.zeros((16,), jnp.float32) for _ in range(4)]
            slot0 = bb * L
            for l in range(L):
                ws, rs, j = (w0, rc0, l) if l < 16 else (w1, rc1, l - 16)
                cidx = jnp.full((16,), j, jnp.int32)
                wl = ws.at[cidx].get(mode="promise_in_bounds")
                rc = rs.at[cidx].get(mode="promise_in_bounds")
                rsplat = jnp.full((16,), 0, jnp.int32) + (slot0 + l)
                col = rc + lane
                g0 = plsc.load_gather(rows_v, [rsplat, col])
                g1 = plsc.load_gather(rows_v, [rsplat, col + 16])
                p = 2 * (l % 2)
                acc[p] = acc[p] + wl * g0
                acc[p + 1] = acc[p + 1] + wl * g1
            obase = (ci % 2) * C + bb
            orow = jnp.full((16,), 0, jnp.int32) + (obase // RPS)
            ocol0 = (obase % RPS) * D
            plsc.store_scatter(out_v, [orow, ocol0 + lane], acc[0] + acc[2])
            plsc.store_scatter(out_v, [orow, ocol0 + 16 + lane], acc[1] + acc[3])
            return carry

        lax.fori_loop(0, C, b_body, 0)

        @pl.when(ci % 2 == 1)
        def _():
            base = pl.multiple_of(
                wid * (PER_W * D // 128) + (ci // 2) * OROWS, OROWS
            )
            pltpu.sync_copy(out_v, out_hbm.at[pl.ds(base, OROWS)])

    stage(0, 0)

    def outer(c2, carry):
        for b in range(2):
            ci = c2 * 2 + b

            @pl.when(ci + 1 < NCHUNK)
            def _():
                stage(ci + 1, 1 - b)

            drain(b)
            combine(ci, b)
        return carry

    lax.fori_loop(0, NCHUNK // 2, outer, 0)


SB = 2048
NI = 123
P = NI * SB
VPAD = RPS * P


def _tr_body(x0, x1, x2, x3, out_ref):
    eye = jnp.eye(D, dtype=jnp.float32)
    dn = (((0,), (0,)), ((), ()))

    def tr(x):
        return jax.lax.dot_general(
            x[...], eye, dn,
            precision=jax.lax.Precision.HIGHEST,
            preferred_element_type=jnp.float32,
        )

    out_ref[...] = jnp.concatenate([tr(x0), tr(x1), tr(x2), tr(x3)], axis=1)


_tr_call = pl.pallas_call(
    _tr_body,
    grid=(NI,),
    in_specs=[
        pl.BlockSpec((D, SB), functools.partial(lambda j, i: (0, j * NI + i), j))
        for j in range(RPS)
    ],
    out_specs=pl.BlockSpec((SB, 128), lambda i: (i, 0)),
    out_shape=jax.ShapeDtypeStruct((P, 128), jnp.float32),
)


_embed_call = functools.partial(
    pl.kernel,
    mesh=plsc.VectorSubcoreMesh(core_axis_name="c", subcore_axis_name="s"),
    compiler_params=pltpu.CompilerParams(
        needs_layout_passes=False, use_tc_tiling_on_sc=True
    ),
    out_type=jax.ShapeDtypeStruct((B * D // 128, 128), jnp.float32),
    scratch_types=[
        pltpu.VMEM((IDX_PER_CHUNK,), jnp.int32),
        pltpu.VMEM((IDX_PER_CHUNK,), jnp.int32),
        pltpu.VMEM((IDX_PER_CHUNK, 128), jnp.float32),
        pltpu.VMEM((IDX_PER_CHUNK, 128), jnp.float32),
        pltpu.VMEM((C * SPAD,), jnp.float32),
        pltpu.VMEM((C * SPAD,), jnp.float32),
        pltpu.VMEM((C * SPAD,), jnp.int32),
        pltpu.VMEM((C * SPAD,), jnp.int32),
        pltpu.VMEM((OROWS, 128), jnp.float32),
        pltpu.SemaphoreType.DMA,
        pltpu.SemaphoreType.DMA,
    ],
)(_embed_body)


@jax.jit
def kernel(indices, weights, table):
    idx32 = indices.astype(jnp.int32)
    q = (idx32 % P).reshape(NW, NCHUNK, 1, IDX_PER_CHUNK)
    rc = jnp.pad((idx32 // P) * D, ((0, 0), (0, SPAD - L)))
    rc = rc.reshape(NW, NCHUNK, 1, C * SPAD)
    w = jnp.pad(weights.astype(jnp.float32), ((0, 0), (0, SPAD - L)))
    w = w.reshape(NW, NCHUNK, 1, C * SPAD)
    tt = jnp.pad(table, ((0, VPAD - V), (0, 0))).T
    tb = _tr_call(tt, tt, tt, tt)
    out = _embed_call(q, w, rc, tb)
    return out.reshape(B, D)


# speedup vs baseline: 1.9748x; 1.4001x over previous
import functools

import jax
import jax.numpy as jnp
from jax import lax
from jax.experimental import pallas as pl
from jax.experimental.pallas import tpu as pltpu
from jax.experimental.pallas import tpu_sc as plsc

B = 16384
L = 20
D = 32
NC = 2
NS = 16
NW = NC * NS
PER_W = B // NW
C = 64
NCHUNK = PER_W // C
IDX_PER_CHUNK = C * L


def _embed_body(
    idx_hbm, w_hbm, table_hbm, out_hbm,
    idx_v0, idx_v1, rows_v0, rows_v1, w_v0, w_v1, out_v, sem0, sem1,
):
    cid = lax.axis_index("c")
    sid = lax.axis_index("s")
    wid = cid * NS + sid

    idx_b = (idx_v0, idx_v1)
    rows_b = (rows_v0, rows_v1)
    w_b = (w_v0, w_v1)
    sems = (sem0, sem1)

    lane = lax.iota(jnp.int32, 16)

    def stage(ci, b):
        base = wid * PER_W + ci * C
        pltpu.sync_copy(idx_hbm.at[:, pl.ds(base, C)], idx_b[b])
        for l in range(L):
            pltpu.async_copy(
                table_hbm.at[idx_b[b].at[l]],
                rows_b[b].at[pl.ds(l * C, C)],
                sems[b],
            )
        pltpu.sync_copy(w_hbm.at[:, pl.ds(base, C)], w_b[b])

    def drain(b):
        for l in range(L):
            pltpu.make_async_copy(
                table_hbm.at[idx_b[b].at[l]],
                rows_b[b].at[pl.ds(l * C, C)],
                sems[b],
            ).wait()

    def combine(ci, b):
        rows_v = rows_b[b]
        w_v = w_b[b]

        lo_l = lane
        hi_l = lane % 4 + 16

        def b_body(bb, carry):
            bsplat = jnp.full((16,), 0, jnp.int32) + bb
            wcol0 = plsc.load_gather(w_v, [lo_l, bsplat])
            wcol1 = plsc.load_gather(w_v, [hi_l, bsplat])
            acc0 = jnp.zeros((16,), jnp.float32)
            acc1 = jnp.zeros((16,), jnp.float32)
            for l in range(L):
                src = wcol0 if l < 16 else wcol1
                wl = src.at[jnp.full((16,), l % 16 if l < 16 else l % 4, jnp.int32)].get(
                    mode="promise_in_bounds"
                )
                r = l * C + bb
                acc0 = acc0 + wl * rows_v[r, pl.ds(0, 16)]
                acc1 = acc1 + wl * rows_v[r, pl.ds(16, 16)]
            out_v[bb, pl.ds(0, 16)] = acc0
            out_v[bb, pl.ds(16, 16)] = acc1
            return carry

        lax.fori_loop(0, C, b_body, 0)
        base = wid * PER_W + ci * C
        pltpu.sync_copy(out_v, out_hbm.at[pl.ds(base, C)])

    stage(0, 0)

    def outer(c2, carry):
        for b in range(2):
            ci = c2 * 2 + b

            @pl.when(ci + 1 < NCHUNK)
            def _():
                stage(ci + 1, 1 - b)

            drain(b)
            combine(ci, b)
        return carry

    lax.fori_loop(0, NCHUNK // 2, outer, 0)


_embed_call = functools.partial(
    pl.kernel,
    mesh=plsc.VectorSubcoreMesh(core_axis_name="c", subcore_axis_name="s"),
    compiler_params=pltpu.CompilerParams(
        needs_layout_passes=False, use_tc_tiling_on_sc=False
    ),
    out_type=jax.ShapeDtypeStruct((B, D), jnp.float32),
    scratch_types=[
        pltpu.VMEM((L, C), jnp.int32),
        pltpu.VMEM((L, C), jnp.int32),
        pltpu.VMEM((IDX_PER_CHUNK, D), jnp.float32),
        pltpu.VMEM((IDX_PER_CHUNK, D), jnp.float32),
        pltpu.VMEM((L, C), jnp.float32),
        pltpu.VMEM((L, C), jnp.float32),
        pltpu.VMEM((C, D), jnp.float32),
        pltpu.SemaphoreType.DMA,
        pltpu.SemaphoreType.DMA,
    ],
)(_embed_body)


@jax.jit
def kernel(indices, weights, table):
    idx_t = indices.astype(jnp.int32).T
    w_t = weights.astype(jnp.float32).T
    return _embed_call(idx_t, w_t, table)
